# baseline (speedup 1.0000x reference)
import jax
import jax.numpy as jnp
from jax import lax
from jax.experimental import pallas as pl
from jax.experimental.pallas import tpu as pltpu
from jax.experimental.pallas import tpu_sc as plsc

VOCAB = 100000
D_MODEL = 1024
BATCH = 4
SEQ = 4096
SCALE = 32.0

NC = 2
NS = 16
NW = NC * NS
LANES = 16

POS_PER_W = SEQ // NW
PC = 8
NCHUNK = POS_PER_W // PC
NG = 3
ROWS = BATCH * PC
VPR = D_MODEL // LANES


def _sc_body(x_hbm, pe_hbm, table_hbm, out_hbm, *scr):
    idxa = scr[0]
    pe_v = scr[1:3]
    tb = scr[3:3 + NG]
    g = scr[3 + NG:3 + 2 * NG]
    o = scr[3 + 2 * NG:3 + 2 * NG + NG * BATCH]
    q = scr[3 + 2 * NG + NG * BATCH:3 + 2 * NG + NG * BATCH + 2]

    wid = lax.axis_index("s") * NC + lax.axis_index("c")
    pos_base = wid * POS_PER_W

    pltpu.async_copy(x_hbm.at[pl.ds(wid * NCHUNK, NCHUNK)], idxa,
                     o[0]).wait()

    def start_gather(c):
        slot = c % NG
        return pltpu.async_copy(table_hbm.at[idxa.at[c]], tb[slot], g[slot])

    def start_pe(c):
        return pltpu.async_copy(pe_hbm.at[pl.ds(pos_base + c * PC, PC)],
                                pe_v[c % 2], q[c % 2])

    pending = {("q", 0): start_pe(0), ("q", 1): start_pe(1)}
    for c in range(NG):
        pending[("g", c)] = start_gather(c)

    for c in range(NCHUNK):
        grp = c % NG
        pending.pop(("g", c)).wait()
        pending.pop(("q", c)).wait()
        buf = tb[grp]
        pe_b = pe_v[c % 2]

        @plsc.parallel_loop(0, PC * VPR, step=1, unroll=4)
        def fma_body(v, buf=buf, pe_b=pe_b):
            r = v >> 6
            sl = pl.ds(pl.multiple_of((v << 4) & (D_MODEL - 1), LANES), LANES)
            pv = pe_b[r, sl]
            for b in range(BATCH):
                buf[b * PC + r, sl] = buf[b * PC + r, sl] * SCALE + pv

        if c + 2 < NCHUNK:
            pending[("q", c + 2)] = start_pe(c + 2)
        for b in range(BATCH):
            row0 = b * SEQ + pos_base + c * PC
            pending[("o", c, b)] = pltpu.async_copy(
                buf.at[pl.ds(b * PC, PC)], out_hbm.at[pl.ds(row0, PC)],
                o[grp * BATCH + b])
        cn = c + NG
        if cn < NCHUNK:
            for b in range(BATCH):
                pending.pop(("o", c, b)).wait()
            pending[("g", cn)] = start_gather(cn)

    for c in range(NCHUNK - NG, NCHUNK):
        for b in range(BATCH):
            pending.pop(("o", c, b)).wait()


@jax.jit
def _embed(x_t, table, pe_flat):
    mesh = plsc.VectorSubcoreMesh(core_axis_name="c", subcore_axis_name="s")
    out = pl.kernel(
        _sc_body,
        out_type=jax.ShapeDtypeStruct((BATCH * SEQ, D_MODEL), jnp.float32),
        mesh=mesh,
        scratch_types=(
            [pltpu.VMEM((NCHUNK, ROWS), jnp.int32)]
            + [pltpu.VMEM((PC, D_MODEL), jnp.float32) for _ in range(2)]
            + [pltpu.VMEM((ROWS, D_MODEL), jnp.float32) for _ in range(NG)]
            + [pltpu.SemaphoreType.DMA for _ in range(NG + NG * BATCH + 2)]
        ),
    )(x_t, pe_flat, table)
    return out


def kernel(x, table, pe):
    x_t = (x.astype(jnp.int32)
           .reshape(BATCH, NW, NCHUNK, PC)
           .transpose(1, 2, 0, 3)
           .reshape(NW * NCHUNK, ROWS))
    pe_flat = pe.reshape(-1, D_MODEL)[:SEQ]
    out = _embed(x_t, table, pe_flat)
    return out.reshape(BATCH, SEQ, D_MODEL)
